# baseline (speedup 1.0000x reference)
import functools
import math

import jax
import jax.numpy as jnp
from jax import lax
from jax.experimental import pallas as pl
from jax.experimental.pallas import tpu as pltpu
from jax.experimental.pallas import tpu_sc as plsc

B, C, T, V, Q, H = 4, 64, 1024, 8192, 256, 8
DH = C // H
VB = 512
NV = V // VB
P_POOL = B * Q
P_FULL = B * T
P = P_POOL + P_FULL
EPS = 1e-5
FBIG = 1e9

NC, NS = 2, 16
NW = NC * NS
BPW = P_POOL // NW
FPW = P_FULL // NW
SPW = V // NS


def _rms_cols(x, g_col):
    pieces = []
    for h in range(H):
        xh = x[h * DH:(h + 1) * DH, :]
        ss = jnp.sum(xh * xh, axis=0, keepdims=True)
        inv = lax.rsqrt(ss * (1.0 / DH) + EPS)
        pieces.append(jnp.broadcast_to(inv, xh.shape))
    return x * jnp.concatenate(pieces, axis=0) * g_col


def _expand_head_rows(c):
    return jnp.concatenate(
        [jnp.broadcast_to(c[h:h + 1, :], (DH, c.shape[1])) for h in range(H)],
        axis=0)


def _interp_matrix():
    t = lax.broadcasted_iota(jnp.int32, (Q, T), 1).astype(jnp.float32)
    s = lax.broadcasted_iota(jnp.int32, (Q, T), 0).astype(jnp.float32)
    src = jnp.clip((t + 0.5) * (Q / T) - 0.5, 0.0, Q - 1.0)
    i0 = jnp.floor(src)
    w = src - i0
    i1 = jnp.minimum(i0 + 1.0, Q - 1.0)
    return (s == i0) * (1.0 - w) + (s == i1) * w


def _argmax_kernel(z_btc, cb_blk, WqT, bq, WkT, bk, WvT, bv, WpT, bp, gq_col,
                   gk_col, idx_out, value_out, wqT_s, best_s, bidx_s, la_s,
                   lb_s):
    i = pl.program_id(0)

    @pl.when(i == 0)
    def _prep_queries():
        z4 = z_btc[...].reshape(B, Q, T // Q, C)
        pooled = ((z4[:, :, 0, :] + z4[:, :, 1, :]) + z4[:, :, 2, :]
                  + z4[:, :, 3, :]) * (Q / T)
        hs = jnp.concatenate(
            [pooled.reshape(P_POOL, C), z_btc[...].reshape(P_FULL, C)], axis=0)
        qv = jnp.dot(hs, WqT[...], preferred_element_type=jnp.float32) + bq[...]
        c = jnp.dot(hs, WpT[...], preferred_element_type=jnp.float32) + bp[...]
        qvrT = _rms_cols(jnp.transpose(qv, (1, 0)), gq_col[...])
        wqT_s[...] = _expand_head_rows(jnp.transpose(c, (1, 0))) * qvrT

    @pl.when(i < NV)
    def _compute_tile():
        kk = jnp.dot(cb_blk[...], WkT[...],
                     preferred_element_type=jnp.float32) + bk[...]
        kvfT = _rms_cols(jnp.transpose(kk, (1, 0)), gk_col[...])
        logits = lax.dot_general(kvfT, wqT_s[...], (((0,), (0,)), ((), ())),
                                 preferred_element_type=jnp.float32)

        @pl.when(i % 2 == 0)
        def _():
            la_s[...] = logits

        @pl.when(i % 2 == 1)
        def _():
            lb_s[...] = logits

        val = jnp.dot(cb_blk[...], WvT[...],
                      preferred_element_type=jnp.float32) + bv[...]
        value_out[...] = jnp.concatenate(
            [val, jnp.zeros((VB, 128 - C), jnp.float32)], axis=1)

    @pl.when(i > 0)
    def _scan_prev():
        j = i - 1

        def _scan(logits):
            tmax = jnp.max(logits, axis=0, keepdims=True)
            riota = lax.broadcasted_iota(jnp.int32, (VB, P), 0).astype(
                jnp.float32)
            cand = jnp.where(logits == jnp.broadcast_to(tmax, (VB, P)),
                             riota, FBIG)
            targ = jnp.min(cand, axis=0, keepdims=True) + (j * VB).astype(
                jnp.float32)

            @pl.when(j == 0)
            def _():
                best_s[...] = tmax
                bidx_s[...] = targ

            @pl.when(j > 0)
            def _():
                upd = tmax > best_s[...]
                best_s[...] = jnp.where(upd, tmax, best_s[...])
                bidx_s[...] = jnp.where(upd, targ, bidx_s[...])

        @pl.when(j % 2 == 0)
        def _():
            _scan(la_s[...])

        @pl.when(j % 2 == 1)
        def _():
            _scan(lb_s[...])

    @pl.when(i == NV)
    def _():
        idx_out[...] = bidx_s[...].astype(jnp.int32)


def _sc_gather_bincount(value_hbm, idxp_hbm, idxf_hbm, zq_hbm, counts_hbm,
                        idxp_v, rows_v, idxf_v, ones_v, zslice_v, counts_sh,
                        sem):
    cid = lax.axis_index("c")
    sid = lax.axis_index("s")
    wid = sid * NC + cid
    pltpu.sync_copy(idxp_hbm.at[pl.ds(wid * BPW, BPW)], idxp_v)
    pltpu.async_copy(value_hbm.at[idxp_v], rows_v, sem).wait()
    pltpu.sync_copy(rows_v, zq_hbm.at[pl.ds(wid * BPW, BPW)])
    @pl.loop(0, SPW // 16)
    def _zero(k):
        zslice_v[pl.ds(k * 16, 16)] = jnp.zeros((16,), jnp.float32)

    @pl.loop(0, FPW // 16)
    def _ones(k):
        ones_v[pl.ds(k * 16, 16)] = jnp.full((16,), 1.0, jnp.float32)

    pltpu.sync_copy(zslice_v, counts_sh.at[pl.ds(sid * SPW, SPW)])
    plsc.subcore_barrier()
    pltpu.sync_copy(idxf_hbm.at[pl.ds(wid * FPW, FPW)], idxf_v)
    pltpu.sync_copy(ones_v, counts_sh.at[idxf_v], add=True)
    plsc.subcore_barrier()
    pltpu.sync_copy(counts_sh.at[pl.ds(sid * SPW, SPW)], zslice_v)
    pltpu.sync_copy(zslice_v, counts_hbm.at[cid, pl.ds(sid * SPW, SPW)])


def _finish_kernel(zq, counts2, zhat_out, perp_out):
    zqT = jnp.transpose(zq[:, 0:C], (1, 0))
    M = _interp_matrix()
    for b in range(B):
        zhat_out[b, :, :] = jnp.dot(zqT[:, b * Q:(b + 1) * Q], M,
                                    preferred_element_type=jnp.float32)
    counts = (counts2[0:1, :] + counts2[1:2, :]).reshape(V // 128, 128)
    p = counts * (1.0 / P_FULL)
    plog = jnp.sum(p * jnp.log(p + 1e-7), axis=(0, 1), keepdims=True)
    perp_out[...] = jnp.exp(-plog)


@functools.partial(jax.jit, static_argnames=())
def _run(z, codebook, Wq, bq, Wk, bk, Wv, bv, Wp, bp, gq, gk):
    z_btc = jnp.transpose(z, (0, 2, 1))
    idx, value = pl.pallas_call(
        _argmax_kernel,
        grid=(NV + 1,),
        in_specs=[
            pl.BlockSpec((B, T, C), lambda i: (0, 0, 0)),
            pl.BlockSpec((VB, C), lambda i: (jnp.minimum(i, NV - 1), 0)),
            pl.BlockSpec((C, C), lambda i: (0, 0)),
            pl.BlockSpec((1, C), lambda i: (0, 0)),
            pl.BlockSpec((C, C), lambda i: (0, 0)),
            pl.BlockSpec((1, C), lambda i: (0, 0)),
            pl.BlockSpec((C, C), lambda i: (0, 0)),
            pl.BlockSpec((1, C), lambda i: (0, 0)),
            pl.BlockSpec((C, H), lambda i: (0, 0)),
            pl.BlockSpec((1, H), lambda i: (0, 0)),
            pl.BlockSpec((C, 1), lambda i: (0, 0)),
            pl.BlockSpec((C, 1), lambda i: (0, 0)),
        ],
        out_specs=[
            pl.BlockSpec((1, P), lambda i: (0, 0)),
            pl.BlockSpec((VB, 128), lambda i: (jnp.minimum(i, NV - 1), 0)),
        ],
        out_shape=[
            jax.ShapeDtypeStruct((1, P), jnp.int32),
            jax.ShapeDtypeStruct((V, 128), jnp.float32),
        ],
        scratch_shapes=[
            pltpu.VMEM((C, P), jnp.float32),
            pltpu.VMEM((1, P), jnp.float32),
            pltpu.VMEM((1, P), jnp.float32),
            pltpu.VMEM((VB, P), jnp.float32),
            pltpu.VMEM((VB, P), jnp.float32),
        ],
    )(z_btc, codebook, Wq.T, bq.reshape(1, C), Wk.T, bk.reshape(1, C), Wv.T,
      bv.reshape(1, C), Wp.T, bp.reshape(1, H),
      jnp.tile(gq, H).reshape(C, 1), jnp.tile(gk, H).reshape(C, 1))

    idx_flat = idx.reshape(P)
    idxp = idx_flat[:P_POOL]
    idxf = idx_flat[P_POOL:]

    mesh = plsc.VectorSubcoreMesh(core_axis_name="c", subcore_axis_name="s")
    zq, counts2 = pl.kernel(
        _sc_gather_bincount,
        mesh=mesh,
        out_type=[
            jax.ShapeDtypeStruct((P_POOL, 128), jnp.float32),
            jax.ShapeDtypeStruct((NC, V), jnp.float32),
        ],
        scratch_types=[
            pltpu.VMEM((BPW,), jnp.int32),
            pltpu.VMEM((BPW, 128), jnp.float32),
            pltpu.VMEM((FPW,), jnp.int32),
            pltpu.VMEM((FPW,), jnp.float32),
            pltpu.VMEM((SPW,), jnp.float32),
            pltpu.VMEM_SHARED((V,), jnp.float32),
            pltpu.SemaphoreType.DMA,
        ],
    )(value, idxp, idxf)

    zhat, perp = pl.pallas_call(
        _finish_kernel,
        grid=(1,),
        in_specs=[
            pl.BlockSpec((P_POOL, 128), lambda i: (0, 0)),
            pl.BlockSpec((NC, V), lambda i: (0, 0)),
        ],
        out_specs=[
            pl.BlockSpec((B, C, T), lambda i: (0, 0, 0)),
            pl.BlockSpec((1, 1), lambda i: (0, 0)),
        ],
        out_shape=[
            jax.ShapeDtypeStruct((B, C, T), jnp.float32),
            jax.ShapeDtypeStruct((1, 1), jnp.float32),
        ],
    )(zq, counts2)
    return zhat, perp[0, 0]


def kernel(z, q, codebook, Wq, bq, Wk, bk, Wv, bv, Wp, bp, gq, gk):
    del q
    return _run(z, codebook, Wq, bq, Wk, bk, Wv, bv, Wp, bp, gq, gk)


# speedup vs baseline: 1.4287x; 1.4287x over previous
import functools

import jax
import jax.numpy as jnp
from jax import lax
from jax.experimental import pallas as pl
from jax.experimental.pallas import tpu as pltpu
from jax.experimental.pallas import tpu_sc as plsc

B, C, T, V, Q, H = 4, 64, 1024, 8192, 256, 8
DH = C // H
VB = 2048
NV = V // VB
P_POOL = B * Q
P_FULL = B * T
P = P_POOL + P_FULL
EPS = 1e-5
FBIG = 1e9

NC, NS = 2, 16
NW = NC * NS
BPW = P_POOL // NW
FPW = P_FULL // NW
SPW = V // NS


def _rms_cols(x, g_col):
    pieces = []
    for h in range(H):
        xh = x[h * DH:(h + 1) * DH, :]
        ss = jnp.sum(xh * xh, axis=0, keepdims=True)
        inv = lax.rsqrt(ss * (1.0 / DH) + EPS)
        pieces.append(jnp.broadcast_to(inv, xh.shape))
    return x * jnp.concatenate(pieces, axis=0) * g_col


def _expand_head_rows(c):
    return jnp.concatenate(
        [jnp.broadcast_to(c[h:h + 1, :], (DH, c.shape[1])) for h in range(H)],
        axis=0)


def _interp_matrix():
    t = lax.broadcasted_iota(jnp.int32, (Q, T), 1).astype(jnp.float32)
    s = lax.broadcasted_iota(jnp.int32, (Q, T), 0).astype(jnp.float32)
    src = jnp.clip((t + 0.5) * (Q / T) - 0.5, 0.0, Q - 1.0)
    i0 = jnp.floor(src)
    w = src - i0
    i1 = jnp.minimum(i0 + 1.0, Q - 1.0)
    return (s == i0) * (1.0 - w) + (s == i1) * w


def _argmax_kernel(z_btc, cb_blk, WqT, bq, WkT, bk, WvT, bv, WpT, bp, gq_col,
                   gk_col, idx_out, value_out, wqT_s, best_s, bidx_s):
    i = pl.program_id(0)

    @pl.when(i == 0)
    def _prep_queries():
        z4 = z_btc[...].reshape(B, Q, T // Q, C)
        pooled = ((z4[:, :, 0, :] + z4[:, :, 1, :]) + z4[:, :, 2, :]
                  + z4[:, :, 3, :]) * (Q / T)
        hs = jnp.concatenate(
            [pooled.reshape(P_POOL, C), z_btc[...].reshape(P_FULL, C)], axis=0)
        qv = jnp.dot(hs, WqT[...], preferred_element_type=jnp.float32) + bq[...]
        c = jnp.dot(hs, WpT[...], preferred_element_type=jnp.float32) + bp[...]
        qvrT = _rms_cols(jnp.transpose(qv, (1, 0)), gq_col[...])
        wqT_s[...] = _expand_head_rows(jnp.transpose(c, (1, 0))) * qvrT

    kk = jnp.dot(cb_blk[...], WkT[...],
                 preferred_element_type=jnp.float32) + bk[...]
    kvfT = _rms_cols(jnp.transpose(kk, (1, 0)), gk_col[...])
    logits = lax.dot_general(kvfT, wqT_s[...], (((0,), (0,)), ((), ())),
                             preferred_element_type=jnp.float32)
    tmax = jnp.max(logits, axis=0, keepdims=True)
    riota = lax.broadcasted_iota(jnp.int32, (VB, P), 0).astype(jnp.float32)
    cand = jnp.where(logits == jnp.broadcast_to(tmax, (VB, P)), riota, FBIG)
    targ = jnp.min(cand, axis=0, keepdims=True) + (i * VB).astype(jnp.float32)

    @pl.when(i == 0)
    def _():
        best_s[...] = tmax
        bidx_s[...] = targ

    @pl.when(i > 0)
    def _():
        upd = tmax > best_s[...]
        best_s[...] = jnp.where(upd, tmax, best_s[...])
        bidx_s[...] = jnp.where(upd, targ, bidx_s[...])

    val = jnp.dot(cb_blk[...], WvT[...],
                  preferred_element_type=jnp.float32) + bv[...]
    value_out[...] = jnp.concatenate(
        [val, jnp.zeros((VB, 128 - C), jnp.float32)], axis=1)

    @pl.when(i == NV - 1)
    def _():
        idx_out[...] = bidx_s[...].astype(jnp.int32)


def _sc_gather_bincount(value_hbm, idxp_hbm, idxf_hbm, zq_hbm, counts_hbm,
                        idxp_v, rows_v, idxf_v, ones_v, zslice_v, counts_sh,
                        sem):
    cid = lax.axis_index("c")
    sid = lax.axis_index("s")
    wid = sid * NC + cid
    pltpu.sync_copy(idxp_hbm.at[pl.ds(wid * BPW, BPW)], idxp_v)
    pltpu.async_copy(value_hbm.at[idxp_v], rows_v, sem).wait()
    pltpu.sync_copy(rows_v, zq_hbm.at[pl.ds(wid * BPW, BPW)])
    @pl.loop(0, SPW // 16)
    def _zero(k):
        zslice_v[pl.ds(k * 16, 16)] = jnp.zeros((16,), jnp.float32)

    @pl.loop(0, FPW // 16)
    def _ones(k):
        ones_v[pl.ds(k * 16, 16)] = jnp.full((16,), 1.0, jnp.float32)

    pltpu.sync_copy(zslice_v, counts_sh.at[pl.ds(sid * SPW, SPW)])
    plsc.subcore_barrier()
    pltpu.sync_copy(idxf_hbm.at[pl.ds(wid * FPW, FPW)], idxf_v)
    pltpu.sync_copy(ones_v, counts_sh.at[idxf_v], add=True)
    plsc.subcore_barrier()
    pltpu.sync_copy(counts_sh.at[pl.ds(sid * SPW, SPW)], zslice_v)
    pltpu.sync_copy(zslice_v, counts_hbm.at[cid, pl.ds(sid * SPW, SPW)])


def _finish_kernel(zq, counts2, zhat_out, perp_out):
    zqT = jnp.transpose(zq[:, 0:C], (1, 0))
    M = _interp_matrix()
    for b in range(B):
        zhat_out[b, :, :] = jnp.dot(zqT[:, b * Q:(b + 1) * Q], M,
                                    preferred_element_type=jnp.float32)
    counts = (counts2[0:1, :] + counts2[1:2, :]).reshape(V // 128, 128)
    p = counts * (1.0 / P_FULL)
    plog = jnp.sum(p * jnp.log(p + 1e-7), axis=(0, 1), keepdims=True)
    perp_out[...] = jnp.exp(-plog)


@functools.partial(jax.jit, static_argnames=())
def _run(z, codebook, Wq, bq, Wk, bk, Wv, bv, Wp, bp, gq, gk):
    z_btc = jnp.transpose(z, (0, 2, 1))
    idx, value = pl.pallas_call(
        _argmax_kernel,
        grid=(NV,),
        in_specs=[
            pl.BlockSpec((B, T, C), lambda i: (0, 0, 0)),
            pl.BlockSpec((VB, C), lambda i: (i, 0)),
            pl.BlockSpec((C, C), lambda i: (0, 0)),
            pl.BlockSpec((1, C), lambda i: (0, 0)),
            pl.BlockSpec((C, C), lambda i: (0, 0)),
            pl.BlockSpec((1, C), lambda i: (0, 0)),
            pl.BlockSpec((C, C), lambda i: (0, 0)),
            pl.BlockSpec((1, C), lambda i: (0, 0)),
            pl.BlockSpec((C, H), lambda i: (0, 0)),
            pl.BlockSpec((1, H), lambda i: (0, 0)),
            pl.BlockSpec((C, 1), lambda i: (0, 0)),
            pl.BlockSpec((C, 1), lambda i: (0, 0)),
        ],
        out_specs=[
            pl.BlockSpec((1, P), lambda i: (0, 0)),
            pl.BlockSpec((VB, 128), lambda i: (i, 0)),
        ],
        out_shape=[
            jax.ShapeDtypeStruct((1, P), jnp.int32),
            jax.ShapeDtypeStruct((V, 128), jnp.float32),
        ],
        scratch_shapes=[
            pltpu.VMEM((C, P), jnp.float32),
            pltpu.VMEM((1, P), jnp.float32),
            pltpu.VMEM((1, P), jnp.float32),
        ],
    )(z_btc, codebook, Wq.T, bq.reshape(1, C), Wk.T, bk.reshape(1, C), Wv.T,
      bv.reshape(1, C), Wp.T, bp.reshape(1, H),
      jnp.tile(gq, H).reshape(C, 1), jnp.tile(gk, H).reshape(C, 1))

    idx_flat = idx.reshape(P)
    idxp = idx_flat[:P_POOL]
    idxf = idx_flat[P_POOL:]

    mesh = plsc.VectorSubcoreMesh(core_axis_name="c", subcore_axis_name="s")
    zq, counts2 = pl.kernel(
        _sc_gather_bincount,
        mesh=mesh,
        out_type=[
            jax.ShapeDtypeStruct((P_POOL, 128), jnp.float32),
            jax.ShapeDtypeStruct((NC, V), jnp.float32),
        ],
        scratch_types=[
            pltpu.VMEM((BPW,), jnp.int32),
            pltpu.VMEM((BPW, 128), jnp.float32),
            pltpu.VMEM((FPW,), jnp.int32),
            pltpu.VMEM((FPW,), jnp.float32),
            pltpu.VMEM((SPW,), jnp.float32),
            pltpu.VMEM_SHARED((V,), jnp.float32),
            pltpu.SemaphoreType.DMA,
        ],
    )(value, idxp, idxf)

    zhat, perp = pl.pallas_call(
        _finish_kernel,
        grid=(1,),
        in_specs=[
            pl.BlockSpec((P_POOL, 128), lambda i: (0, 0)),
            pl.BlockSpec((NC, V), lambda i: (0, 0)),
        ],
        out_specs=[
            pl.BlockSpec((B, C, T), lambda i: (0, 0, 0)),
            pl.BlockSpec((1, 1), lambda i: (0, 0)),
        ],
        out_shape=[
            jax.ShapeDtypeStruct((B, C, T), jnp.float32),
            jax.ShapeDtypeStruct((1, 1), jnp.float32),
        ],
    )(zq, counts2)
    return zhat, perp[0, 0]


def kernel(z, q, codebook, Wq, bq, Wk, bk, Wv, bv, Wp, bp, gq, gk):
    del q
    return _run(z, codebook, Wq, bq, Wk, bk, Wv, bv, Wp, bp, gq, gk)
